# Initial kernel scaffold; baseline (speedup 1.0000x reference)
import jax
import jax.numpy as jnp
from jax.experimental import pallas as pl


def kernel(x, edge_index, W1, b1, g1, bt1, W2, b2, g2, bt2, W3, b3, g3, bt3):
    raise NotImplementedError("write your pallas kernel here")


# speedup vs baseline: 23.7489x; 23.7489x over previous
import functools

import jax
import jax.numpy as jnp
from jax import lax
from jax.experimental import pallas as pl
from jax.experimental.pallas import tpu as pltpu
from jax.experimental.pallas import tpu_sc as plsc

L = 512
E = 8192
NTILES = 32
EPT = E // NTILES
SLICE = (L * L) // 16


def _sc_edge_hist(edges_flat):
    mesh = plsc.VectorSubcoreMesh(core_axis_name="c", subcore_axis_name="s")

    @functools.partial(
        pl.kernel,
        mesh=mesh,
        out_type=jax.ShapeDtypeStruct((2, L * L), jnp.float32),
        scratch_types=[
            pltpu.VMEM((EPT,), jnp.int32),
            pltpu.VMEM((EPT,), jnp.int32),
            pltpu.VMEM((2, 128), jnp.int32),
            pltpu.VMEM((2, 128), jnp.float32),
            pltpu.VMEM((SLICE,), jnp.float32),
            pltpu.VMEM_SHARED((L * L,), jnp.float32),
        ],
    )
    def hist(edges_hbm, out_hbm, rows_v, cols_v, idx_v, val_v, zb_v, acc_sh):
        cid = lax.axis_index("c")
        sid = lax.axis_index("s")
        tid = cid * 16 + sid

        pltpu.sync_copy(edges_hbm.at[pl.ds(tid * EPT, EPT)], rows_v)
        pltpu.sync_copy(edges_hbm.at[pl.ds(E + tid * EPT, EPT)], cols_v)

        def zero_body(i, _):
            zb_v[pl.ds(i * 16, 16)] = jnp.zeros((16,), jnp.float32)
            return 0

        lax.fori_loop(0, SLICE // 16, zero_body, 0)
        pltpu.sync_copy(zb_v, acc_sh.at[pl.ds(sid * SLICE, SLICE)])

        for k in range(EPT // 16):
            r = rows_v[pl.ds(k * 16, 16)]
            c = cols_v[pl.ds(k * 16, 16)]
            idx_v[k // 8, pl.ds((k % 8) * 16, 16)] = r * L + c
            val_v[k // 8, pl.ds((k % 8) * 16, 16)] = jnp.full((16,), 1.0, jnp.float32)

        plsc.subcore_barrier()

        for j in range(2):
            pltpu.sync_copy(val_v.at[j], acc_sh.at[idx_v.at[j]], add=True)

        plsc.subcore_barrier()

        pltpu.sync_copy(acc_sh.at[pl.ds(sid * SLICE, SLICE)],
                        out_hbm.at[cid, pl.ds(sid * SLICE, SLICE)])

    return hist(edges_flat)


def _tc_body(xc_hbm, sp_ref, wt1_ref, b1_ref, g1_ref, bt1_ref,
             wt2_ref, b2_ref, g2_ref, bt2_ref,
             wt3_ref, b3_ref, g3_ref, bt3_ref, out_hbm, h_scr, sem):
    pltpu.make_async_copy(xc_hbm, h_scr, sem).start()

    S = sp_ref[0] + sp_ref[1]
    ir = lax.broadcasted_iota(jnp.int32, (L, L), 0)
    ic = lax.broadcasted_iota(jnp.int32, (L, L), 1)
    S = jnp.where(ir == ic, S + 2.0, S)
    deg = jnp.sum(S, axis=0, keepdims=True)
    dinv = lax.rsqrt(deg)
    Sn = S * dinv

    pltpu.make_async_copy(xc_hbm, h_scr, sem).wait()

    params = ((wt1_ref, b1_ref, g1_ref, bt1_ref),
              (wt2_ref, b2_ref, g2_ref, bt2_ref),
              (wt3_ref, b3_ref, g3_ref, bt3_ref))
    for wt_ref, b_ref, g_ref, bt_ref in params:
        ssum = jnp.zeros((32, 1, 1), jnp.float32)
        ssq = jnp.zeros((32, 1, 1), jnp.float32)
        for bg in range(8):
            sl = pl.ds(bg * 32, 32)
            hm = h_scr[:, sl, :].reshape(32, 32 * L)
            t1 = jnp.dot(wt_ref[...], hm,
                         preferred_element_type=jnp.float32,
                         precision=lax.Precision.HIGHEST)
            t1 = t1.reshape(32, 32, L).reshape(32 * 32, L)
            t2 = jnp.dot(t1 * dinv, Sn,
                         preferred_element_type=jnp.float32,
                         precision=lax.Precision.HIGHEST)
            u = t2.reshape(32, 32, L) + b_ref[...]
            u = u - jnp.mean(u, axis=1, keepdims=True)
            ssum = ssum + jnp.sum(u, axis=(1, 2), keepdims=True)
            ssq = ssq + jnp.sum(u * u, axis=(1, 2), keepdims=True)
            h_scr[:, sl, :] = u
        m = ssum * (1.0 / (256 * L))
        v = ssq * (1.0 / (256 * L)) - m * m
        scale = g_ref[...] * lax.rsqrt(v + 1e-5)
        shift = bt_ref[...] - m * scale
        for bg in range(8):
            sl = pl.ds(bg * 32, 32)
            h_scr[:, sl, :] = jnp.maximum(h_scr[:, sl, :] * scale + shift, 0.0)

    pltpu.make_async_copy(h_scr, out_hbm, sem).start()
    pltpu.make_async_copy(h_scr, out_hbm, sem).wait()


def _tc_call(xc, sp, args, interpret=False):
    any_spec = pl.BlockSpec(memory_space=pl.ANY)
    return pl.pallas_call(
        _tc_body,
        out_shape=jax.ShapeDtypeStruct((32, 256, L), jnp.float32),
        in_specs=[any_spec] + [pl.BlockSpec(memory_space=pltpu.MemorySpace.VMEM)] * 13,
        out_specs=any_spec,
        scratch_shapes=[
            pltpu.VMEM((32, 256, L), jnp.float32),
            pltpu.SemaphoreType.DMA,
        ],
        interpret=interpret,
    )(xc, sp, *args)


def kernel(x, edge_index, W1, b1, g1, bt1, W2, b2, g2, bt2, W3, b3, g3, bt3):
    xc = x.reshape(256, 32, L).transpose(1, 0, 2)
    sp = _sc_edge_hist(edge_index.reshape(2 * E)).reshape(2, L, L)

    def v3(a):
        return a.reshape(32, 1, 1)

    yc = _tc_call(xc, sp, (W1.T, v3(b1), v3(g1), v3(bt1),
                           W2.T, v3(b2), v3(g2), v3(bt2),
                           W3.T, v3(b3), v3(g3), v3(bt3)))
    return yc.transpose(1, 0, 2)


# speedup vs baseline: 26.8690x; 1.1314x over previous
import functools

import jax
import jax.numpy as jnp
from jax import lax
from jax.experimental import pallas as pl
from jax.experimental.pallas import tpu as pltpu
from jax.experimental.pallas import tpu_sc as plsc

L = 512
E = 8192
NTILES = 32
EPT = E // NTILES
SLICE = (L * L) // 16


def _sc_edge_hist(edges_flat):
    mesh = plsc.VectorSubcoreMesh(core_axis_name="c", subcore_axis_name="s")

    @functools.partial(
        pl.kernel,
        mesh=mesh,
        out_type=jax.ShapeDtypeStruct((2, L * L), jnp.float32),
        scratch_types=[
            pltpu.VMEM((EPT,), jnp.int32),
            pltpu.VMEM((EPT,), jnp.int32),
            pltpu.VMEM((2, 128), jnp.int32),
            pltpu.VMEM((2, 128), jnp.float32),
            pltpu.VMEM((SLICE,), jnp.float32),
            pltpu.VMEM_SHARED((L * L,), jnp.float32),
        ],
    )
    def hist(edges_hbm, out_hbm, rows_v, cols_v, idx_v, val_v, zb_v, acc_sh):
        cid = lax.axis_index("c")
        sid = lax.axis_index("s")
        tid = cid * 16 + sid

        pltpu.sync_copy(edges_hbm.at[pl.ds(tid * EPT, EPT)], rows_v)
        pltpu.sync_copy(edges_hbm.at[pl.ds(E + tid * EPT, EPT)], cols_v)

        def zero_body(i, _):
            zb_v[pl.ds(i * 16, 16)] = jnp.zeros((16,), jnp.float32)
            return 0

        lax.fori_loop(0, SLICE // 16, zero_body, 0)
        pltpu.sync_copy(zb_v, acc_sh.at[pl.ds(sid * SLICE, SLICE)])

        for k in range(EPT // 16):
            r = rows_v[pl.ds(k * 16, 16)]
            c = cols_v[pl.ds(k * 16, 16)]
            idx_v[k // 8, pl.ds((k % 8) * 16, 16)] = r * L + c
            val_v[k // 8, pl.ds((k % 8) * 16, 16)] = jnp.full((16,), 1.0, jnp.float32)

        plsc.subcore_barrier()

        for j in range(2):
            pltpu.sync_copy(val_v.at[j], acc_sh.at[idx_v.at[j]], add=True)

        plsc.subcore_barrier()

        pltpu.sync_copy(acc_sh.at[pl.ds(sid * SLICE, SLICE)],
                        out_hbm.at[cid, pl.ds(sid * SLICE, SLICE)])

    return hist(edges_flat)


def _tc_body(x_hbm, sp_ref, wt1_ref, b1_ref, g1_ref, bt1_ref,
             wt2_ref, b2_ref, g2_ref, bt2_ref,
             wt3_ref, b3_ref, g3_ref, bt3_ref, out_hbm, h_scr, sem):
    in_copies = [pltpu.make_async_copy(x_hbm.at[:, c, :], h_scr.at[c], sem)
                 for c in range(32)]
    for cp in in_copies:
        cp.start()

    S = sp_ref[0] + sp_ref[1]
    ir = lax.broadcasted_iota(jnp.int32, (L, L), 0)
    ic = lax.broadcasted_iota(jnp.int32, (L, L), 1)
    S = jnp.where(ir == ic, S + 2.0, S)
    deg = jnp.sum(S, axis=0, keepdims=True)
    dinv = lax.rsqrt(deg)
    Sn = S * dinv

    for cp in in_copies:
        cp.wait()

    params = ((wt1_ref, b1_ref, g1_ref, bt1_ref),
              (wt2_ref, b2_ref, g2_ref, bt2_ref),
              (wt3_ref, b3_ref, g3_ref, bt3_ref))
    for li, (wt_ref, b_ref, g_ref, bt_ref) in enumerate(params):
        ssum = jnp.zeros((32, 1, 1), jnp.float32)
        ssq = jnp.zeros((32, 1, 1), jnp.float32)
        for bg in range(8):
            sl = pl.ds(bg * 32, 32)
            hm = h_scr[:, sl, :].reshape(32, 32 * L)
            t1 = jnp.dot(wt_ref[...], hm,
                         preferred_element_type=jnp.float32,
                         precision=lax.Precision.HIGHEST)
            t1 = t1.reshape(32, 32, L).reshape(32 * 32, L)
            t2 = jnp.dot(t1 * dinv, Sn,
                         preferred_element_type=jnp.float32,
                         precision=lax.Precision.HIGHEST)
            u = t2.reshape(32, 32, L) + b_ref[...]
            u = u - jnp.mean(u, axis=1, keepdims=True)
            ssum = ssum + jnp.sum(u, axis=(1, 2), keepdims=True)
            ssq = ssq + jnp.sum(u * u, axis=(1, 2), keepdims=True)
            h_scr[:, sl, :] = u
        m = ssum * (1.0 / (256 * L))
        v = ssq * (1.0 / (256 * L)) - m * m
        scale = g_ref[...] * lax.rsqrt(v + 1e-5)
        shift = bt_ref[...] - m * scale
        if li < 2:
            for bg in range(8):
                sl = pl.ds(bg * 32, 32)
                h_scr[:, sl, :] = jnp.maximum(h_scr[:, sl, :] * scale + shift, 0.0)
        else:
            out_copies = []
            for c0 in range(0, 32, 8):
                cs = pl.ds(c0, 8)
                h_scr[cs] = jnp.maximum(
                    h_scr[cs] * scale[c0:c0 + 8] + shift[c0:c0 + 8], 0.0)
                for c in range(c0, c0 + 8):
                    cp = pltpu.make_async_copy(h_scr.at[c], out_hbm.at[:, c, :], sem)
                    cp.start()
                    out_copies.append(cp)
            for cp in out_copies:
                cp.wait()


def _tc_call(xc, sp, args, interpret=False):
    any_spec = pl.BlockSpec(memory_space=pl.ANY)
    return pl.pallas_call(
        _tc_body,
        out_shape=jax.ShapeDtypeStruct((256, 32, L), jnp.float32),
        in_specs=[any_spec] + [pl.BlockSpec(memory_space=pltpu.MemorySpace.VMEM)] * 13,
        out_specs=any_spec,
        scratch_shapes=[
            pltpu.VMEM((32, 256, L), jnp.float32),
            pltpu.SemaphoreType.DMA,
        ],
        interpret=interpret,
    )(xc, sp, *args)


def kernel(x, edge_index, W1, b1, g1, bt1, W2, b2, g2, bt2, W3, b3, g3, bt3):
    sp = _sc_edge_hist(edge_index.reshape(2 * E)).reshape(2, L, L)

    def v3(a):
        return a.reshape(32, 1, 1)

    return _tc_call(x.reshape(256, 32, L), sp,
                    (W1.T, v3(b1), v3(g1), v3(bt1),
                     W2.T, v3(b2), v3(g2), v3(bt2),
                     W3.T, v3(b3), v3(g3), v3(bt3)))


# speedup vs baseline: 50.2770x; 1.8712x over previous
import functools

import jax
import jax.numpy as jnp
from jax import lax
from jax.experimental import pallas as pl
from jax.experimental.pallas import tpu as pltpu
from jax.experimental.pallas import tpu_sc as plsc

L = 512
E = 8192
NTILES = 32
EPT = E // NTILES
SLICE = (L * L) // 16


def _sc_edge_hist(edges_flat):
    mesh = plsc.VectorSubcoreMesh(core_axis_name="c", subcore_axis_name="s")

    @functools.partial(
        pl.kernel,
        mesh=mesh,
        out_type=jax.ShapeDtypeStruct((2, L * L), jnp.float32),
        scratch_types=[
            pltpu.VMEM((EPT,), jnp.int32),
            pltpu.VMEM((EPT,), jnp.int32),
            pltpu.VMEM((2, 128), jnp.int32),
            pltpu.VMEM((2, 128), jnp.float32),
            pltpu.VMEM((SLICE,), jnp.float32),
            pltpu.VMEM_SHARED((L * L,), jnp.float32),
        ],
    )
    def hist(edges_hbm, out_hbm, rows_v, cols_v, idx_v, val_v, zb_v, acc_sh):
        cid = lax.axis_index("c")
        sid = lax.axis_index("s")
        tid = cid * 16 + sid

        pltpu.sync_copy(edges_hbm.at[pl.ds(tid * EPT, EPT)], rows_v)
        pltpu.sync_copy(edges_hbm.at[pl.ds(E + tid * EPT, EPT)], cols_v)

        def zero_body(i, _):
            zb_v[pl.ds(i * 16, 16)] = jnp.zeros((16,), jnp.float32)
            return 0

        lax.fori_loop(0, SLICE // 16, zero_body, 0)
        pltpu.sync_copy(zb_v, acc_sh.at[pl.ds(sid * SLICE, SLICE)])

        for k in range(EPT // 16):
            r = rows_v[pl.ds(k * 16, 16)]
            c = cols_v[pl.ds(k * 16, 16)]
            idx_v[k // 8, pl.ds((k % 8) * 16, 16)] = r * L + c
            val_v[k // 8, pl.ds((k % 8) * 16, 16)] = jnp.full((16,), 1.0, jnp.float32)

        plsc.subcore_barrier()

        for j in range(2):
            pltpu.sync_copy(val_v.at[j], acc_sh.at[idx_v.at[j]], add=True)

        plsc.subcore_barrier()

        pltpu.sync_copy(acc_sh.at[pl.ds(sid * SLICE, SLICE)],
                        out_hbm.at[cid, pl.ds(sid * SLICE, SLICE)])

    return hist(edges_flat)


def _tc_body(x_hbm, sp_ref, wt1_ref, b1_ref, g1_ref, bt1_ref,
             wt2_ref, b2_ref, g2_ref, bt2_ref,
             wt3_ref, b3_ref, g3_ref, bt3_ref, out_hbm, h_scr, sem):
    in_copies = [pltpu.make_async_copy(x_hbm.at[:, c, :], h_scr.at[c], sem)
                 for c in range(32)]
    for cp in in_copies:
        cp.start()

    S = sp_ref[0] + sp_ref[1]
    ir = lax.broadcasted_iota(jnp.int32, (L, L), 0)
    ic = lax.broadcasted_iota(jnp.int32, (L, L), 1)
    S = jnp.where(ir == ic, S + 2.0, S)
    deg = jnp.sum(S, axis=0, keepdims=True)
    dinv = lax.rsqrt(deg)
    Sn = S * dinv

    for cp in in_copies:
        cp.wait()

    params = ((wt1_ref, b1_ref, g1_ref, bt1_ref),
              (wt2_ref, b2_ref, g2_ref, bt2_ref),
              (wt3_ref, b3_ref, g3_ref, bt3_ref))
    pending = None
    for li, (wt_ref, b_ref, g_ref, bt_ref) in enumerate(params):
        ssum = jnp.zeros((32, 1, 1), jnp.float32)
        ssq = jnp.zeros((32, 1, 1), jnp.float32)
        for bg in range(8):
            sl = pl.ds(bg * 32, 32)
            hv = h_scr[:, sl, :]
            if pending is not None:
                hv = jnp.maximum(hv * pending[0] + pending[1], 0.0)
            hm = hv.reshape(32, 32 * L)
            t1 = jnp.dot(wt_ref[...], hm,
                         preferred_element_type=jnp.float32,
                         precision=lax.Precision.DEFAULT)
            t1 = t1.reshape(32, 32, L).reshape(32 * 32, L)
            t2 = jnp.dot(t1 * dinv, Sn,
                         preferred_element_type=jnp.float32,
                         precision=lax.Precision.DEFAULT)
            u = t2.reshape(32, 32, L) + b_ref[...]
            u = u - jnp.mean(u, axis=1, keepdims=True)
            ssum = ssum + jnp.sum(u, axis=(1, 2), keepdims=True)
            ssq = ssq + jnp.sum(u * u, axis=(1, 2), keepdims=True)
            h_scr[:, sl, :] = u
        m = ssum * (1.0 / (256 * L))
        v = ssq * (1.0 / (256 * L)) - m * m
        scale = g_ref[...] * lax.rsqrt(v + 1e-5)
        shift = bt_ref[...] - m * scale
        if li < 2:
            pending = (scale, shift)
        else:
            out_copies = []
            for c0 in range(0, 32, 8):
                cs = pl.ds(c0, 8)
                h_scr[cs] = jnp.maximum(
                    h_scr[cs] * scale[c0:c0 + 8] + shift[c0:c0 + 8], 0.0)
                for c in range(c0, c0 + 8):
                    cp = pltpu.make_async_copy(h_scr.at[c], out_hbm.at[:, c, :], sem)
                    cp.start()
                    out_copies.append(cp)
            for cp in out_copies:
                cp.wait()


def _tc_call(xc, sp, args, interpret=False):
    any_spec = pl.BlockSpec(memory_space=pl.ANY)
    return pl.pallas_call(
        _tc_body,
        out_shape=jax.ShapeDtypeStruct((256, 32, L), jnp.float32),
        in_specs=[any_spec] + [pl.BlockSpec(memory_space=pltpu.MemorySpace.VMEM)] * 13,
        out_specs=any_spec,
        scratch_shapes=[
            pltpu.VMEM((32, 256, L), jnp.float32),
            pltpu.SemaphoreType.DMA,
        ],
        interpret=interpret,
    )(xc, sp, *args)


def kernel(x, edge_index, W1, b1, g1, bt1, W2, b2, g2, bt2, W3, b3, g3, bt3):
    sp = _sc_edge_hist(edge_index.reshape(2 * E)).reshape(2, L, L)

    def v3(a):
        return a.reshape(32, 1, 1)

    return _tc_call(x.reshape(256, 32, L), sp,
                    (W1.T, v3(b1), v3(g1), v3(bt1),
                     W2.T, v3(b2), v3(g2), v3(bt2),
                     W3.T, v3(b3), v3(g3), v3(bt3)))


# speedup vs baseline: 56.2671x; 1.1191x over previous
import functools

import jax
import jax.numpy as jnp
from jax import lax
from jax.experimental import pallas as pl
from jax.experimental.pallas import tpu as pltpu
from jax.experimental.pallas import tpu_sc as plsc

L = 512
E = 8192
NTILES = 32
EPT = E // NTILES
SLICE = (L * L) // 16


def _sc_edge_hist(edges_flat):
    mesh = plsc.VectorSubcoreMesh(core_axis_name="c", subcore_axis_name="s")

    @functools.partial(
        pl.kernel,
        mesh=mesh,
        out_type=jax.ShapeDtypeStruct((2, L * L), jnp.float32),
        scratch_types=[
            pltpu.VMEM((EPT,), jnp.int32),
            pltpu.VMEM((EPT,), jnp.int32),
            pltpu.VMEM((2, 128), jnp.int32),
            pltpu.VMEM((2, 128), jnp.float32),
            pltpu.VMEM((SLICE,), jnp.float32),
            pltpu.VMEM_SHARED((L * L,), jnp.float32),
        ],
    )
    def hist(edges_hbm, out_hbm, rows_v, cols_v, idx_v, val_v, zb_v, acc_sh):
        cid = lax.axis_index("c")
        sid = lax.axis_index("s")
        tid = cid * 16 + sid

        pltpu.sync_copy(edges_hbm.at[pl.ds(tid * EPT, EPT)], rows_v)
        pltpu.sync_copy(edges_hbm.at[pl.ds(E + tid * EPT, EPT)], cols_v)

        def zero_body(i, _):
            zb_v[pl.ds(i * 16, 16)] = jnp.zeros((16,), jnp.float32)
            return 0

        lax.fori_loop(0, SLICE // 16, zero_body, 0)
        pltpu.sync_copy(zb_v, acc_sh.at[pl.ds(sid * SLICE, SLICE)])

        for k in range(EPT // 16):
            r = rows_v[pl.ds(k * 16, 16)]
            c = cols_v[pl.ds(k * 16, 16)]
            idx_v[k // 8, pl.ds((k % 8) * 16, 16)] = r * L + c
            val_v[k // 8, pl.ds((k % 8) * 16, 16)] = jnp.full((16,), 1.0, jnp.float32)

        plsc.subcore_barrier()

        for j in range(2):
            pltpu.sync_copy(val_v.at[j], acc_sh.at[idx_v.at[j]], add=True)

        plsc.subcore_barrier()

        pltpu.sync_copy(acc_sh.at[pl.ds(sid * SLICE, SLICE)],
                        out_hbm.at[cid, pl.ds(sid * SLICE, SLICE)])

    return hist(edges_flat)


def _tc_body(x_hbm, sp_ref, wt1_ref, g1_ref, bt1_ref,
             wt2_ref, g2_ref, bt2_ref,
             wt3_ref, g3_ref, bt3_ref, out_hbm, h_scr, sem):
    in_copies = [pltpu.make_async_copy(x_hbm.at[:, c, :], h_scr.at[c], sem)
                 for c in range(32)]
    for cp in in_copies:
        cp.start()

    S = sp_ref[0] + sp_ref[1]
    ir = lax.broadcasted_iota(jnp.int32, (L, L), 0)
    ic = lax.broadcasted_iota(jnp.int32, (L, L), 1)
    eye = ir == ic
    S = jnp.where(eye, S + 2.0, S)
    deg = jnp.sum(S, axis=0, keepdims=True)
    dinv = lax.rsqrt(deg)
    D = jnp.where(eye, dinv, 0.0)
    At = jnp.dot(D, S * dinv,
                 preferred_element_type=jnp.float32,
                 precision=lax.Precision.DEFAULT)

    for cp in in_copies:
        cp.wait()

    CG = 64
    NCH = 256 // CG
    params = ((wt1_ref, g1_ref, bt1_ref),
              (wt2_ref, g2_ref, bt2_ref),
              (wt3_ref, g3_ref, bt3_ref))
    pending = None
    for li, (wt_ref, g_ref, bt_ref) in enumerate(params):
        ssq = jnp.zeros((32, 1, 1), jnp.float32)
        for bg in range(NCH):
            sl = pl.ds(bg * CG, CG)
            hv = h_scr[:, sl, :]
            if pending is not None:
                hv = jnp.maximum(hv * pending[0] + pending[1], 0.0)
            hm = hv.reshape(32, CG * L)
            t1 = jnp.dot(wt_ref[...], hm,
                         preferred_element_type=jnp.float32,
                         precision=lax.Precision.DEFAULT)
            t1 = t1.reshape(32, CG, L).reshape(32 * CG, L)
            t2 = jnp.dot(t1, At,
                         preferred_element_type=jnp.float32,
                         precision=lax.Precision.DEFAULT)
            u4 = t2.reshape(32, CG // 32, 32, L)
            u4 = u4 - jnp.mean(u4, axis=2, keepdims=True)
            u = u4.reshape(32, CG, L)
            ssq = ssq + jnp.sum(u * u, axis=(1, 2), keepdims=True)
            h_scr[:, sl, :] = u
        v = ssq * (1.0 / (256 * L))
        scale = g_ref[...] * lax.rsqrt(v + 1e-5)
        shift = bt_ref[...]
        if li < 2:
            pending = (scale, shift)
        else:
            out_copies = []
            for c0 in range(0, 32, 8):
                cs = pl.ds(c0, 8)
                h_scr[cs] = jnp.maximum(
                    h_scr[cs] * scale[c0:c0 + 8] + shift[c0:c0 + 8], 0.0)
                for c in range(c0, c0 + 8):
                    cp = pltpu.make_async_copy(h_scr.at[c], out_hbm.at[:, c, :], sem)
                    cp.start()
                    out_copies.append(cp)
            for cp in out_copies:
                cp.wait()


def _tc_call(xc, sp, args, interpret=False):
    any_spec = pl.BlockSpec(memory_space=pl.ANY)
    return pl.pallas_call(
        _tc_body,
        out_shape=jax.ShapeDtypeStruct((256, 32, L), jnp.float32),
        in_specs=[any_spec] + [pl.BlockSpec(memory_space=pltpu.MemorySpace.VMEM)] * 10,
        out_specs=any_spec,
        scratch_shapes=[
            pltpu.VMEM((32, 256, L), jnp.float32),
            pltpu.SemaphoreType.DMA,
        ],
        interpret=interpret,
    )(xc, sp, *args)


def kernel(x, edge_index, W1, b1, g1, bt1, W2, b2, g2, bt2, W3, b3, g3, bt3):
    sp = _sc_edge_hist(edge_index.reshape(2 * E)).reshape(2, L, L)

    def v3(a):
        return a.reshape(32, 1, 1)

    return _tc_call(x.reshape(256, 32, L), sp,
                    (W1.T, v3(g1), v3(bt1),
                     W2.T, v3(g2), v3(bt2),
                     W3.T, v3(g3), v3(bt3)))
